# baseline (speedup 1.0000x reference)
import functools

import jax
import jax.numpy as jnp
from jax import lax
from jax.experimental import pallas as pl
from jax.experimental.pallas import tpu as pltpu
from jax.experimental.pallas import tpu_sc as plsc

N = 10000
E = 320000
D = 128
H = 64
G = 64
OUT = 5

NP = 10240
K = 128
C = 2560
EP = C * K
NW = 32
CPT0 = 144
CPT1 = 16
CPTM = max(CPT0, CPT1)
RPT = NP // 16
RB = 1024
GRID = NP // RB


def _make_sc_agg(count: bool):
    out_type = [jax.ShapeDtypeStruct((2, NP, H), jnp.float32)]
    scratch = [
        pltpu.VMEM((CPTM, 2, K), jnp.int32),
        pltpu.VMEM((K, H), jnp.float32),
        pltpu.VMEM((K, H), jnp.float32),
        pltpu.VMEM_SHARED((NP, H), jnp.float32),
        pltpu.SemaphoreType.DMA,
    ]
    if count:
        out_type.append(jax.ShapeDtypeStruct((2, NP, 16), jnp.float32))
        scratch.insert(3, pltpu.VMEM((K, 16), jnp.float32))
        scratch.insert(4, pltpu.VMEM_SHARED((NP, 16), jnp.float32))

    mesh = plsc.VectorSubcoreMesh(core_axis_name="c", subcore_axis_name="s",
                                  num_cores=2, num_subcores=16)

    def body(y_hbm, edges_hbm, *refs):
        if count:
            (out_sum, out_cnt, idx_v, rows0_v, rows1_v, ones_v, cnt_sh,
             acc_sh, sem) = refs
        else:
            out_sum, idx_v, rows0_v, rows1_v, acc_sh, sem = refs
        rows = (rows0_v, rows1_v)
        cid = lax.axis_index("c")
        sid = lax.axis_index("s")
        row0 = sid * RPT
        nc = jnp.where(cid == 0, CPT0, CPT1)
        start = jnp.where(cid == 0, sid * CPT0, 16 * CPT0 + sid * CPT1)

        if CPT0 > 0:
            @pl.when(cid == 0)
            def _():
                pltpu.async_copy(edges_hbm.at[pl.ds(start, CPT0)],
                                 idx_v.at[pl.ds(0, CPT0)], sem)
        if CPT1 > 0:
            @pl.when(cid == 1)
            def _():
                pltpu.async_copy(edges_hbm.at[pl.ds(start, CPT1)],
                                 idx_v.at[pl.ds(0, CPT1)], sem)

        z16 = jnp.zeros((16,), jnp.float32)

        def zrow(j, _):
            for c4 in range(H // 16):
                rows0_v[j, pl.ds(c4 * 16, 16)] = z16
            return ()
        lax.fori_loop(0, K, zrow, ())
        for k in range(RPT // K):
            pltpu.sync_copy(rows0_v, acc_sh.at[pl.ds(row0 + k * K, K)])
        if count:
            def zcnt(j, _):
                ones_v[j, :] = z16
                return ()
            lax.fori_loop(0, K, zcnt, ())
            for k in range(RPT // K):
                pltpu.sync_copy(ones_v, cnt_sh.at[pl.ds(row0 + k * K, K)])

            def frow(j, _):
                ones_v[j, :] = jnp.ones((16,), jnp.float32)
                return ()
            lax.fori_loop(0, K, frow, ())
        if CPT0 > 0:
            @pl.when(cid == 0)
            def _():
                pltpu.make_async_copy(edges_hbm.at[pl.ds(start, CPT0)],
                                      idx_v.at[pl.ds(0, CPT0)], sem).wait()
        if CPT1 > 0:
            @pl.when(cid == 1)
            def _():
                pltpu.make_async_copy(edges_hbm.at[pl.ds(start, CPT1)],
                                      idx_v.at[pl.ds(0, CPT1)], sem).wait()
        plsc.subcore_barrier()

        @pl.when(nc > 0)
        def _():
            pltpu.async_copy(y_hbm.at[idx_v.at[0, 0]], rows0_v, sem)

        def pair(p, _):
            for b in range(2):
                j = p * 2 + b
                pltpu.make_async_copy(y_hbm.at[idx_v.at[j, 0]], rows[b],
                                      sem).wait()

                @pl.when(j + 1 < nc)
                def _():
                    pltpu.async_copy(y_hbm.at[idx_v.at[j + 1, 0]],
                                     rows[1 - b], sem)
                pltpu.sync_copy(rows[b], acc_sh.at[idx_v.at[j, 1]], add=True)
                if count:
                    pltpu.sync_copy(ones_v, cnt_sh.at[idx_v.at[j, 1]],
                                    add=True)
            return ()
        lax.fori_loop(0, nc // 2, pair, ())
        plsc.subcore_barrier()

        for k in range(RPT // K):
            sl = pl.ds(row0 + k * K, K)
            pltpu.sync_copy(acc_sh.at[sl], rows0_v)
            pltpu.sync_copy(rows0_v, out_sum.at[cid, sl])
            if count:
                pltpu.sync_copy(cnt_sh.at[sl], ones_v)
                pltpu.sync_copy(ones_v, out_cnt.at[cid, sl])

    return pl.kernel(body, out_type=tuple(out_type), mesh=mesh,
                     scratch_types=scratch,
                     compiler_params=pltpu.CompilerParams(
                         use_tc_tiling_on_sc=False))


@functools.lru_cache(maxsize=None)
def _sc_agg_fn(count: bool):
    return _make_sc_agg(count)


def _dotT(a, w):
    return lax.dot_general(a, w, (((1,), (1,)), ((), ())),
                           preferred_element_type=jnp.float32)


def _k1_body(x_ref, w1l_ref, w1r_ref, b1l_ref, y_ref, r_ref):
    xb = x_ref[...]
    y_ref[...] = _dotT(xb, w1l_ref[...])
    r_ref[...] = _dotT(xb, w1r_ref[...]) + b1l_ref[...]


def _k2_body(sum_ref, cnt_ref, r1_ref, w2l_ref, w2r_ref, b2l_ref,
             y2_ref, r2_ref):
    s = sum_ref[0] + sum_ref[1]
    c = cnt_ref[0, :, 0:1] + cnt_ref[1, :, 0:1]
    h = jnp.maximum(s / jnp.maximum(c, 1.0) + r1_ref[...], 0.0)
    y2_ref[...] = _dotT(h, w2l_ref[...])
    r2_ref[...] = _dotT(h, w2r_ref[...]) + b2l_ref[...]


def _k3_body(sum_ref, cnt_ref, r2_ref, batch_ref, wg_ref, bg_ref,
             wo_ref, bo_ref, out_ref, acc_ref):
    i = pl.program_id(0)

    @pl.when(i == 0)
    def _():
        acc_ref[...] = jnp.zeros_like(acc_ref)

    s = sum_ref[0] + sum_ref[1]
    c = cnt_ref[0, :, 0:1] + cnt_ref[1, :, 0:1]
    h = jnp.maximum(s / jnp.maximum(c, 1.0) + r2_ref[...], 0.0)
    onehot = (batch_ref[...] ==
              lax.broadcasted_iota(jnp.int32, (RB, G), 1)).astype(jnp.float32)
    he = jnp.concatenate([h, jnp.ones((RB, 1), jnp.float32)], axis=1)
    acc_ref[...] += lax.dot_general(onehot, he, (((0,), (0,)), ((), ())),
                                    preferred_element_type=jnp.float32)

    @pl.when(i == GRID - 1)
    def _():
        gs = acc_ref[:, :H]
        gc = acc_ref[:, H:H + 1]
        g = gs / jnp.maximum(gc, 1.0)
        g1 = jnp.maximum(_dotT(g, wg_ref[...]) + bg_ref[...], 0.0)
        out_ref[...] = _dotT(g1, wo_ref[...]) + bo_ref[...]


def _tc_project(x_p, W1l, W1r, b1l):
    return pl.pallas_call(
        _k1_body,
        grid=(GRID,),
        in_specs=[
            pl.BlockSpec((RB, D), lambda i: (i, 0)),
            pl.BlockSpec((H, D), lambda i: (0, 0)),
            pl.BlockSpec((H, D), lambda i: (0, 0)),
            pl.BlockSpec((1, H), lambda i: (0, 0)),
        ],
        out_specs=[
            pl.BlockSpec((RB, H), lambda i: (i, 0)),
            pl.BlockSpec((RB, H), lambda i: (i, 0)),
        ],
        out_shape=[
            jax.ShapeDtypeStruct((NP, H), jnp.float32),
            jax.ShapeDtypeStruct((NP, H), jnp.float32),
        ],
    )(x_p, W1l, W1r, b1l.reshape(1, H))


def _tc_combine(sums, cnts, r1, W2l, W2r, b2l):
    return pl.pallas_call(
        _k2_body,
        grid=(GRID,),
        in_specs=[
            pl.BlockSpec((2, RB, H), lambda i: (0, i, 0)),
            pl.BlockSpec((2, RB, 16), lambda i: (0, i, 0)),
            pl.BlockSpec((RB, H), lambda i: (i, 0)),
            pl.BlockSpec((H, H), lambda i: (0, 0)),
            pl.BlockSpec((H, H), lambda i: (0, 0)),
            pl.BlockSpec((1, H), lambda i: (0, 0)),
        ],
        out_specs=[
            pl.BlockSpec((RB, H), lambda i: (i, 0)),
            pl.BlockSpec((RB, H), lambda i: (i, 0)),
        ],
        out_shape=[
            jax.ShapeDtypeStruct((NP, H), jnp.float32),
            jax.ShapeDtypeStruct((NP, H), jnp.float32),
        ],
    )(sums, cnts, r1, W2l, W2r, b2l.reshape(1, H))


def _tc_pool_mlp(sums, cnts, r2, batch_p, Wg, bg, Wo, bo):
    return pl.pallas_call(
        _k3_body,
        grid=(GRID,),
        in_specs=[
            pl.BlockSpec((2, RB, H), lambda i: (0, i, 0)),
            pl.BlockSpec((2, RB, 16), lambda i: (0, i, 0)),
            pl.BlockSpec((RB, H), lambda i: (i, 0)),
            pl.BlockSpec((RB, 1), lambda i: (i, 0)),
            pl.BlockSpec((G, H), lambda i: (0, 0)),
            pl.BlockSpec((1, G), lambda i: (0, 0)),
            pl.BlockSpec((OUT, G), lambda i: (0, 0)),
            pl.BlockSpec((1, OUT), lambda i: (0, 0)),
        ],
        out_specs=pl.BlockSpec((G, OUT), lambda i: (0, 0)),
        out_shape=jax.ShapeDtypeStruct((G, OUT), jnp.float32),
        scratch_shapes=[pltpu.VMEM((G, H + 1), jnp.float32)],
        compiler_params=pltpu.CompilerParams(
            dimension_semantics=("arbitrary",)),
    )(sums, cnts, r2, batch_p, Wg, bg.reshape(1, G), Wo, bo.reshape(1, OUT))


def kernel(x, edge_index, batch, W1l, b1l, W1r, W2l, b2l, W2r, Wg, bg, Wo, bo):
    x_p = jnp.pad(x, ((0, NP - N), (0, 0)))
    batch_p = jnp.pad(batch, (0, NP - N), constant_values=G).reshape(NP, 1)
    src = jnp.pad(edge_index[0], (0, EP - E), constant_values=0)
    pad_dst = N + (jnp.arange(EP - E, dtype=jnp.int32) % (NP - N))
    dst = jnp.concatenate([edge_index[1], pad_dst])
    edges = jnp.stack([src.reshape(C, K), dst.reshape(C, K)], axis=1)

    y1, r1 = _tc_project(x_p, W1l, W1r, b1l)
    sums1, cnts = _sc_agg_fn(True)(y1, edges)
    y2, r2 = _tc_combine(sums1, cnts, r1, W2l, W2r, b2l)
    (sums2,) = _sc_agg_fn(False)(y2, edges)
    return _tc_pool_mlp(sums2, cnts, r2, batch_p, Wg, bg, Wo, bo)


# speedup vs baseline: 1.0711x; 1.0711x over previous
import functools

import jax
import jax.numpy as jnp
from jax import lax
from jax.experimental import pallas as pl
from jax.experimental.pallas import tpu as pltpu
from jax.experimental.pallas import tpu_sc as plsc

N = 10000
E = 320000
D = 128
H = 64
G = 64
OUT = 5

NP = 10240
K = 128
C = 2560
EP = C * K
NW = 32
CPT0 = 136
CPT1 = 24
CPTM = max(CPT0, CPT1)
RPT = NP // 16
RB = 1024
GRID = NP // RB


def _make_sc_agg(count: bool):
    out_type = [jax.ShapeDtypeStruct((2, NP, H), jnp.float32)]
    scratch = [
        pltpu.VMEM((CPTM, 2, K), jnp.int32),
        pltpu.VMEM((K, H), jnp.float32),
        pltpu.VMEM((K, H), jnp.float32),
        pltpu.VMEM_SHARED((NP, H), jnp.float32),
        pltpu.SemaphoreType.DMA,
    ]
    if count:
        out_type.append(jax.ShapeDtypeStruct((2, NP, 16), jnp.float32))
        scratch.insert(3, pltpu.VMEM((K, 16), jnp.float32))
        scratch.insert(4, pltpu.VMEM_SHARED((NP, 16), jnp.float32))

    mesh = plsc.VectorSubcoreMesh(core_axis_name="c", subcore_axis_name="s",
                                  num_cores=2, num_subcores=16)

    def body(y_hbm, edges_hbm, *refs):
        if count:
            (out_sum, out_cnt, idx_v, rows0_v, rows1_v, ones_v, cnt_sh,
             acc_sh, sem) = refs
        else:
            out_sum, idx_v, rows0_v, rows1_v, acc_sh, sem = refs
        rows = (rows0_v, rows1_v)
        cid = lax.axis_index("c")
        sid = lax.axis_index("s")
        row0 = sid * RPT
        nc = jnp.where(cid == 0, CPT0, CPT1)
        start = jnp.where(cid == 0, sid * CPT0, 16 * CPT0 + sid * CPT1)

        if CPT0 > 0:
            @pl.when(cid == 0)
            def _():
                pltpu.async_copy(edges_hbm.at[pl.ds(start, CPT0)],
                                 idx_v.at[pl.ds(0, CPT0)], sem)
        if CPT1 > 0:
            @pl.when(cid == 1)
            def _():
                pltpu.async_copy(edges_hbm.at[pl.ds(start, CPT1)],
                                 idx_v.at[pl.ds(0, CPT1)], sem)

        z16 = jnp.zeros((16,), jnp.float32)

        def zrow(j, _):
            for c4 in range(H // 16):
                rows0_v[j, pl.ds(c4 * 16, 16)] = z16
            return ()
        lax.fori_loop(0, K, zrow, ())
        for k in range(RPT // K):
            pltpu.sync_copy(rows0_v, acc_sh.at[pl.ds(row0 + k * K, K)])
        if count:
            def zcnt(j, _):
                ones_v[j, :] = z16
                return ()
            lax.fori_loop(0, K, zcnt, ())
            for k in range(RPT // K):
                pltpu.sync_copy(ones_v, cnt_sh.at[pl.ds(row0 + k * K, K)])

            def frow(j, _):
                ones_v[j, :] = jnp.ones((16,), jnp.float32)
                return ()
            lax.fori_loop(0, K, frow, ())
        if CPT0 > 0:
            @pl.when(cid == 0)
            def _():
                pltpu.make_async_copy(edges_hbm.at[pl.ds(start, CPT0)],
                                      idx_v.at[pl.ds(0, CPT0)], sem).wait()
        if CPT1 > 0:
            @pl.when(cid == 1)
            def _():
                pltpu.make_async_copy(edges_hbm.at[pl.ds(start, CPT1)],
                                      idx_v.at[pl.ds(0, CPT1)], sem).wait()
        plsc.subcore_barrier()

        @pl.when(nc > 0)
        def _():
            pltpu.async_copy(y_hbm.at[idx_v.at[0, 0]], rows0_v, sem)

        def pair(p, _):
            for b in range(2):
                j = p * 2 + b
                pltpu.make_async_copy(y_hbm.at[idx_v.at[j, 0]], rows[b],
                                      sem).wait()

                @pl.when(j + 1 < nc)
                def _():
                    pltpu.async_copy(y_hbm.at[idx_v.at[j + 1, 0]],
                                     rows[1 - b], sem)
                pltpu.sync_copy(rows[b], acc_sh.at[idx_v.at[j, 1]], add=True)
                if count:
                    pltpu.sync_copy(ones_v, cnt_sh.at[idx_v.at[j, 1]],
                                    add=True)
            return ()
        lax.fori_loop(0, nc // 2, pair, ())
        plsc.subcore_barrier()

        for k in range(RPT // K):
            sl = pl.ds(row0 + k * K, K)
            pltpu.sync_copy(acc_sh.at[sl], rows0_v)
            pltpu.sync_copy(rows0_v, out_sum.at[cid, sl])
            if count:
                pltpu.sync_copy(cnt_sh.at[sl], ones_v)
                pltpu.sync_copy(ones_v, out_cnt.at[cid, sl])

    return pl.kernel(body, out_type=tuple(out_type), mesh=mesh,
                     scratch_types=scratch,
                     compiler_params=pltpu.CompilerParams(
                         use_tc_tiling_on_sc=False))


@functools.lru_cache(maxsize=None)
def _sc_agg_fn(count: bool):
    return _make_sc_agg(count)


def _dotT(a, w):
    return lax.dot_general(a, w, (((1,), (1,)), ((), ())),
                           preferred_element_type=jnp.float32)


def _k1_body(x_ref, w1l_ref, w1r_ref, b1l_ref, y_ref, r_ref):
    xb = x_ref[...]
    y_ref[...] = _dotT(xb, w1l_ref[...])
    r_ref[...] = _dotT(xb, w1r_ref[...]) + b1l_ref[...]


def _k2_body(sum_ref, cnt_ref, r1_ref, w2l_ref, w2r_ref, b2l_ref,
             y2_ref, r2_ref):
    s = sum_ref[0] + sum_ref[1]
    c = cnt_ref[0, :, 0:1] + cnt_ref[1, :, 0:1]
    h = jnp.maximum(s / jnp.maximum(c, 1.0) + r1_ref[...], 0.0)
    y2_ref[...] = _dotT(h, w2l_ref[...])
    r2_ref[...] = _dotT(h, w2r_ref[...]) + b2l_ref[...]


def _k3_body(sum_ref, cnt_ref, r2_ref, batch_ref, wg_ref, bg_ref,
             wo_ref, bo_ref, out_ref, acc_ref):
    i = pl.program_id(0)

    @pl.when(i == 0)
    def _():
        acc_ref[...] = jnp.zeros_like(acc_ref)

    s = sum_ref[0] + sum_ref[1]
    c = cnt_ref[0, :, 0:1] + cnt_ref[1, :, 0:1]
    h = jnp.maximum(s / jnp.maximum(c, 1.0) + r2_ref[...], 0.0)
    onehot = (batch_ref[...] ==
              lax.broadcasted_iota(jnp.int32, (RB, G), 1)).astype(jnp.float32)
    he = jnp.concatenate([h, jnp.ones((RB, 1), jnp.float32)], axis=1)
    acc_ref[...] += lax.dot_general(onehot, he, (((0,), (0,)), ((), ())),
                                    preferred_element_type=jnp.float32)

    @pl.when(i == GRID - 1)
    def _():
        gs = acc_ref[:, :H]
        gc = acc_ref[:, H:H + 1]
        g = gs / jnp.maximum(gc, 1.0)
        g1 = jnp.maximum(_dotT(g, wg_ref[...]) + bg_ref[...], 0.0)
        out_ref[...] = _dotT(g1, wo_ref[...]) + bo_ref[...]


def _tc_project(x_p, W1l, W1r, b1l):
    return pl.pallas_call(
        _k1_body,
        grid=(GRID,),
        in_specs=[
            pl.BlockSpec((RB, D), lambda i: (i, 0)),
            pl.BlockSpec((H, D), lambda i: (0, 0)),
            pl.BlockSpec((H, D), lambda i: (0, 0)),
            pl.BlockSpec((1, H), lambda i: (0, 0)),
        ],
        out_specs=[
            pl.BlockSpec((RB, H), lambda i: (i, 0)),
            pl.BlockSpec((RB, H), lambda i: (i, 0)),
        ],
        out_shape=[
            jax.ShapeDtypeStruct((NP, H), jnp.float32),
            jax.ShapeDtypeStruct((NP, H), jnp.float32),
        ],
    )(x_p, W1l, W1r, b1l.reshape(1, H))


def _tc_combine(sums, cnts, r1, W2l, W2r, b2l):
    return pl.pallas_call(
        _k2_body,
        grid=(GRID,),
        in_specs=[
            pl.BlockSpec((2, RB, H), lambda i: (0, i, 0)),
            pl.BlockSpec((2, RB, 16), lambda i: (0, i, 0)),
            pl.BlockSpec((RB, H), lambda i: (i, 0)),
            pl.BlockSpec((H, H), lambda i: (0, 0)),
            pl.BlockSpec((H, H), lambda i: (0, 0)),
            pl.BlockSpec((1, H), lambda i: (0, 0)),
        ],
        out_specs=[
            pl.BlockSpec((RB, H), lambda i: (i, 0)),
            pl.BlockSpec((RB, H), lambda i: (i, 0)),
        ],
        out_shape=[
            jax.ShapeDtypeStruct((NP, H), jnp.float32),
            jax.ShapeDtypeStruct((NP, H), jnp.float32),
        ],
    )(sums, cnts, r1, W2l, W2r, b2l.reshape(1, H))


def _tc_pool_mlp(sums, cnts, r2, batch_p, Wg, bg, Wo, bo):
    return pl.pallas_call(
        _k3_body,
        grid=(GRID,),
        in_specs=[
            pl.BlockSpec((2, RB, H), lambda i: (0, i, 0)),
            pl.BlockSpec((2, RB, 16), lambda i: (0, i, 0)),
            pl.BlockSpec((RB, H), lambda i: (i, 0)),
            pl.BlockSpec((RB, 1), lambda i: (i, 0)),
            pl.BlockSpec((G, H), lambda i: (0, 0)),
            pl.BlockSpec((1, G), lambda i: (0, 0)),
            pl.BlockSpec((OUT, G), lambda i: (0, 0)),
            pl.BlockSpec((1, OUT), lambda i: (0, 0)),
        ],
        out_specs=pl.BlockSpec((G, OUT), lambda i: (0, 0)),
        out_shape=jax.ShapeDtypeStruct((G, OUT), jnp.float32),
        scratch_shapes=[pltpu.VMEM((G, H + 1), jnp.float32)],
        compiler_params=pltpu.CompilerParams(
            dimension_semantics=("arbitrary",)),
    )(sums, cnts, r2, batch_p, Wg, bg.reshape(1, G), Wo, bo.reshape(1, OUT))


def kernel(x, edge_index, batch, W1l, b1l, W1r, W2l, b2l, W2r, Wg, bg, Wo, bo):
    x_p = jnp.pad(x, ((0, NP - N), (0, 0)))
    batch_p = jnp.pad(batch, (0, NP - N), constant_values=G).reshape(NP, 1)
    src = jnp.pad(edge_index[0], (0, EP - E), constant_values=0)
    pad_dst = N + (jnp.arange(EP - E, dtype=jnp.int32) % (NP - N))
    dst = jnp.concatenate([edge_index[1], pad_dst])
    edges = jnp.stack([src.reshape(C, K), dst.reshape(C, K)], axis=1)

    y1, r1 = _tc_project(x_p, W1l, W1r, b1l)
    sums1, cnts = _sc_agg_fn(True)(y1, edges)
    y2, r2 = _tc_combine(sums1, cnts, r1, W2l, W2r, b2l)
    (sums2,) = _sc_agg_fn(False)(y2, edges)
    return _tc_pool_mlp(sums2, cnts, r2, batch_p, Wg, bg, Wo, bo)
